# baseline (speedup 1.0000x reference)
import functools

import jax
import jax.numpy as jnp
from jax import lax
from jax.experimental import pallas as pl
from jax.experimental.pallas import tpu as pltpu
from jax.experimental.pallas import tpu_sc as plsc

NUM_EMB = 100000
SEM_DIM = 4
EMB_DIM = 64
PAD = NUM_EMB * SEM_DIM

B, L, LF = 4096, 200, 4
NSEQ = B * L
NFUT = B * LF

NC, NS, LANES = 2, 16, 16
NW = NC * NS

SEQ_PER_W = NSEQ // NW
BATCH_PER_W = SEQ_PER_W // L
FUT_PER_W = NFUT // NW

CHUNK_A = 128
CHUNK_B = L - CHUNK_A
NCHUNKS = 2 * BATCH_PER_W
K = 4
SEQ_PER_SET = 2
NGROUPS = BATCH_PER_W // SEQ_PER_SET
NPAIRS = NGROUPS // 2
VECS_PER_SEQ = 13

FCHUNK = 128
FUT_CHUNKS = FUT_PER_W // FCHUNK

SLAB = 1600
SLAB_PAD = SLAB + LANES
NSLABS = SEQ_PER_W // SLAB
SLAB_SEQS = SLAB // L


def _idx_vec(sem_v, tt_v, off, use_mask, msk_v):
  s = sem_v[pl.ds(off, LANES)]
  t = tt_v[pl.ds(off, LANES)]
  tc = jnp.clip(t, 0, SEM_DIM - 1)
  sc = jnp.clip(s, 0, NUM_EMB - 1)
  idx = tc * NUM_EMB + sc
  keep = (s >= 0) & (s < NUM_EMB)
  if use_mask:
    m = msk_v[pl.ds(off, LANES)]
    keep = keep & (m != 0)
  mf = jnp.where(keep, jnp.float32(1.0), jnp.float32(0.0))
  return idx, mf


def _sc_body(sem_h, tt_h, msk_h, semf_h, ttf_h, table_h,
             out_seq3_h, out_fut_h,
             sem_v, tt_v, msk_v, idx_v, maskf_v, fidx_v,
             rows_a, rows_b, gsem, wsem):
  wid = lax.axis_index("s") * NC + lax.axis_index("c")
  base = wid * SEQ_PER_W
  seq0 = wid * BATCH_PER_W
  basef = wid * FUT_PER_W

  def slab_loop(sidx, carry):
    soff = base + sidx * SLAB
    pltpu.sync_copy(sem_h.at[pl.ds(soff, SLAB)], sem_v.at[pl.ds(0, SLAB)])
    pltpu.sync_copy(tt_h.at[pl.ds(soff, SLAB)], tt_v.at[pl.ds(0, SLAB)])
    pltpu.sync_copy(msk_h.at[pl.ds(soff, SLAB)], msk_v.at[pl.ds(0, SLAB)])

    def seq_loop(c, inner):
      sq = sidx * SLAB_SEQS + c
      for u in range(VECS_PER_SEQ):
        idx, mf = _idx_vec(sem_v, tt_v, c * L + u * LANES, True, msk_v)
        if u < CHUNK_A // LANES:
          idx_v[2 * sq, pl.ds(u * LANES, LANES)] = idx
          maskf_v[2 * sq, pl.ds(u * LANES, LANES)] = mf
        else:
          co = (u - CHUNK_A // LANES) * LANES
          idx_v[2 * sq + 1, pl.ds(co, LANES)] = idx
          maskf_v[2 * sq + 1, pl.ds(co, LANES)] = mf
      return inner
    lax.fori_loop(0, SLAB_SEQS, seq_loop, 0)
    return carry
  lax.fori_loop(0, NSLABS, slab_loop, 0)

  pltpu.sync_copy(semf_h.at[pl.ds(basef, FUT_PER_W)],
                  sem_v.at[pl.ds(0, FUT_PER_W)])
  pltpu.sync_copy(ttf_h.at[pl.ds(basef, FUT_PER_W)],
                  tt_v.at[pl.ds(0, FUT_PER_W)])
  for c in range(FUT_CHUNKS):
    for u in range(FCHUNK // LANES):
      idx, _ = _idx_vec(sem_v, tt_v, c * FCHUNK + u * LANES, False, None)
      fidx_v[c, pl.ds(u * LANES, LANES)] = idx

  def fire_gathers(group, rows_set):
    for b in range(K):
      sb, half = divmod(b, 2)
      size = CHUNK_A if half == 0 else CHUNK_B
      off = 0 if half == 0 else CHUNK_A
      pltpu.async_copy(table_h.at[idx_v.at[group * K + b, pl.ds(0, size)]],
                       rows_set.at[sb, pl.ds(off, size)], gsem)

  def _mask_rows(rows_set, sb, roff0, mv, nrows):
    for rr in range(nrows):
      m = mv[rr]
      roff = roff0 + rr
      for c in range(EMB_DIM // LANES):
        v = rows_set[sb, roff, pl.ds(c * LANES, LANES)]
        rows_set[sb, roff, pl.ds(c * LANES, LANES)] = v * m

  def mask_set(group, rows_set):
    for b in range(K):
      sb, half = divmod(b, 2)
      size = CHUNK_A if half == 0 else CHUNK_B
      off = 0 if half == 0 else CHUNK_A
      cid = group * K + b

      def row_loop(r16, carry):
        mv = maskf_v[cid, pl.ds(r16 * LANES, LANES)]
        _mask_rows(rows_set, sb, off + r16 * LANES, mv, LANES)
        return carry
      lax.fori_loop(0, size // LANES, row_loop, 0)
      if size % LANES:
        t0 = (size // LANES) * LANES
        mv = maskf_v[cid, pl.ds(t0, LANES)]
        _mask_rows(rows_set, sb, off + t0, mv, size % LANES)

  def fire_seq_writes(group, rows_set):
    pltpu.async_copy(rows_set,
                     out_seq3_h.at[pl.ds(seq0 + group * SEQ_PER_SET,
                                         SEQ_PER_SET)], wsem)

  def wait_gathers(rows_set):
    pltpu.make_async_copy(out_seq3_h.at[pl.ds(0, SEQ_PER_SET)], rows_set,
                          gsem).wait()

  def wait_writes(rows_set):
    pltpu.make_async_copy(rows_set, out_seq3_h.at[pl.ds(0, SEQ_PER_SET)],
                          wsem).wait()

  fire_gathers(0, rows_a)

  def pair_loop(g2, carry):
    g_a = 2 * g2

    @pl.when(g2 > 0)
    def _():
      wait_writes(rows_b)
    fire_gathers(g_a + 1, rows_b)
    wait_gathers(rows_a)
    mask_set(g_a, rows_a)
    fire_seq_writes(g_a, rows_a)
    wait_writes(rows_a)

    @pl.when(g_a + 2 < NGROUPS)
    def _():
      fire_gathers(g_a + 2, rows_a)
    wait_gathers(rows_b)
    mask_set(g_a + 1, rows_b)
    fire_seq_writes(g_a + 1, rows_b)
    return carry
  lax.fori_loop(0, NPAIRS, pair_loop, 0)

  def fut_gather(c, rows_set, sb):
    pltpu.async_copy(table_h.at[fidx_v.at[c]],
                     rows_set.at[sb, pl.ds(0, FCHUNK)], gsem)

  def fut_write(c, rows_set, sb):
    pltpu.async_copy(rows_set.at[sb, pl.ds(0, FCHUNK)],
                     out_fut_h.at[pl.ds(basef + c * FCHUNK, FCHUNK)], wsem)

  def wait_fut_gathers(rows_set, n):
    for _ in range(n):
      pltpu.make_async_copy(out_fut_h.at[pl.ds(0, FCHUNK)],
                            rows_set.at[0, pl.ds(0, FCHUNK)], gsem).wait()

  def wait_fut_writes(rows_set, n):
    for _ in range(n):
      pltpu.make_async_copy(rows_set.at[0, pl.ds(0, FCHUNK)],
                            out_fut_h.at[pl.ds(0, FCHUNK)], wsem).wait()

  fut_gather(0, rows_a, 0)
  fut_gather(1, rows_a, 1)
  wait_writes(rows_b)
  fut_gather(2, rows_b, 0)
  fut_gather(3, rows_b, 1)
  wait_fut_gathers(rows_a, 2)
  fut_write(0, rows_a, 0)
  fut_write(1, rows_a, 1)
  wait_fut_gathers(rows_b, 2)
  fut_write(2, rows_b, 0)
  fut_write(3, rows_b, 1)
  wait_fut_writes(rows_a, 4)


@jax.jit
def _run(sem_flat, tt_flat, msk_flat, semf_flat, ttf_flat, table):
  mesh = plsc.VectorSubcoreMesh(core_axis_name="c", subcore_axis_name="s",
                                num_cores=NC, num_subcores=NS)
  f = pl.kernel(
      _sc_body,
      out_type=[
          jax.ShapeDtypeStruct((B, L, EMB_DIM), jnp.float32),
          jax.ShapeDtypeStruct((NFUT, EMB_DIM), jnp.float32),
      ],
      mesh=mesh,
      scratch_types=[
          pltpu.VMEM((SLAB_PAD,), jnp.int32),
          pltpu.VMEM((SLAB_PAD,), jnp.int32),
          pltpu.VMEM((SLAB_PAD,), jnp.int32),
          pltpu.VMEM((NCHUNKS, CHUNK_A), jnp.int32),
          pltpu.VMEM((NCHUNKS, CHUNK_A), jnp.float32),
          pltpu.VMEM((FUT_CHUNKS, FCHUNK), jnp.int32),
          pltpu.VMEM((SEQ_PER_SET, L, EMB_DIM), jnp.float32),
          pltpu.VMEM((SEQ_PER_SET, L, EMB_DIM), jnp.float32),
          pltpu.SemaphoreType.DMA,
          pltpu.SemaphoreType.DMA,
      ],
      compiler_params=pltpu.CompilerParams(use_tc_tiling_on_sc=False),
  )
  return f(sem_flat, tt_flat, msk_flat, semf_flat, ttf_flat, table)


def kernel(sem_ids, token_type_ids, seq_mask, sem_ids_fut, token_type_ids_fut,
           table):
  sem_flat = sem_ids.reshape(-1).astype(jnp.int32)
  tt_flat = token_type_ids.reshape(-1).astype(jnp.int32)
  msk_flat = seq_mask.reshape(-1).astype(jnp.int32)
  semf_flat = sem_ids_fut.reshape(-1).astype(jnp.int32)
  ttf_flat = token_type_ids_fut.reshape(-1).astype(jnp.int32)
  out_seq, out_fut = _run(sem_flat, tt_flat, msk_flat, semf_flat, ttf_flat,
                          table.astype(jnp.float32))
  return (out_seq, out_fut.reshape(B, LF, EMB_DIM))


# speedup vs baseline: 1.0111x; 1.0111x over previous
import functools

import jax
import jax.numpy as jnp
from jax import lax
from jax.experimental import pallas as pl
from jax.experimental.pallas import tpu as pltpu
from jax.experimental.pallas import tpu_sc as plsc

NUM_EMB = 100000
SEM_DIM = 4
EMB_DIM = 64
PAD = NUM_EMB * SEM_DIM

B, L, LF = 4096, 200, 4
NSEQ = B * L
NFUT = B * LF

NC, NS, LANES = 2, 16, 16
NW = NC * NS

SEQ_PER_W = NSEQ // NW
BPW = B // NW
FUT_PER_W = NFUT // NW

ET = EMB_DIM // 8
BT = B // 128
COLV = EMB_DIM // LANES
OUTW = 129

FCHUNK = 128
FUT_CHUNKS = FUT_PER_W // FCHUNK

SLAB = 1600
SLAB_PAD = SLAB + LANES
NSLABS = SEQ_PER_W // SLAB
SLAB_SEQS = SLAB // L
VECS_PER_SEQ = 13


def _idx_vec(sem_v, tt_v, off, use_mask, msk_v):
  s = sem_v[pl.ds(off, LANES)]
  t = tt_v[pl.ds(off, LANES)]
  tc = jnp.clip(t, 0, SEM_DIM - 1)
  sc = jnp.clip(s, 0, NUM_EMB - 1)
  idx = tc * NUM_EMB + sc
  keep = (s >= 0) & (s < NUM_EMB)
  if use_mask:
    m = msk_v[pl.ds(off, LANES)]
    keep = keep & (m != 0)
  mf = jnp.where(keep, jnp.float32(1.0), jnp.float32(0.0))
  return idx, mf


def _sc_body(sem_h, tt_h, msk_h, semf_h, ttf_h, table_h,
             out5_h, out_fut_h,
             sem_v, tt_v, msk_v, idx_v, maskf_v, fidx_v,
             rows0, rows1, out0, out1, gsem0, gsem1, wsem0, wsem1):
  wid = lax.axis_index("s") * NC + lax.axis_index("c")
  base = wid * SEQ_PER_W
  basef = wid * FUT_PER_W
  iota = lax.iota(jnp.int32, LANES)

  def slab_loop(sidx, carry):
    soff = base + sidx * SLAB
    pltpu.sync_copy(sem_h.at[pl.ds(soff, SLAB)], sem_v.at[pl.ds(0, SLAB)])
    pltpu.sync_copy(tt_h.at[pl.ds(soff, SLAB)], tt_v.at[pl.ds(0, SLAB)])
    pltpu.sync_copy(msk_h.at[pl.ds(soff, SLAB)], msk_v.at[pl.ds(0, SLAB)])

    def seq_loop(c, inner):
      bcol = sidx * SLAB_SEQS + c
      bvec = jnp.full((LANES,), bcol, jnp.int32)
      for u in range(VECS_PER_SEQ):
        idx, mf = _idx_vec(sem_v, tt_v, c * L + u * LANES, True, msk_v)
        lrow = iota + (u * LANES)
        if u < VECS_PER_SEQ - 1:
          plsc.store_scatter(idx_v, [lrow, bvec], idx)
          plsc.store_scatter(maskf_v, [lrow, bvec], mf)
        else:
          tail = iota < (L - (VECS_PER_SEQ - 1) * LANES)
          plsc.store_scatter(idx_v, [lrow, bvec], idx, mask=tail)
          plsc.store_scatter(maskf_v, [lrow, bvec], mf, mask=tail)
      return inner
    lax.fori_loop(0, SLAB_SEQS, seq_loop, 0)
    return carry
  lax.fori_loop(0, NSLABS, slab_loop, 0)

  pltpu.sync_copy(semf_h.at[pl.ds(basef, FUT_PER_W)],
                  sem_v.at[pl.ds(0, FUT_PER_W)])
  pltpu.sync_copy(ttf_h.at[pl.ds(basef, FUT_PER_W)],
                  tt_v.at[pl.ds(0, FUT_PER_W)])
  for c in range(FUT_CHUNKS):
    for u in range(FCHUNK // LANES):
      idx, _ = _idx_vec(sem_v, tt_v, c * FCHUNK + u * LANES, False, None)
      fidx_v[c, pl.ds(u * LANES, LANES)] = idx

  def fire_gather(l, rows, gsem):
    pltpu.async_copy(table_h.at[idx_v.at[l]], rows, gsem)

  def wait_gather(rows, gsem):
    pltpu.make_async_copy(table_h.at[pl.ds(0, BPW)], rows, gsem).wait()

  def fire_writes(l, out_l, wsem):
    for et in range(ET):
      pltpu.async_copy(out_l.at[pl.ds(et * 8, 8), pl.ds(0, 128)],
                       out5_h.at[l, et, wid], wsem)

  def wait_writes(out_l, wsem):
    for et in range(ET):
      pltpu.make_async_copy(out_l.at[pl.ds(et * 8, 8), pl.ds(0, 128)],
                            out5_h.at[0, et, 0], wsem).wait()

  def transpose_mask(l, rows, out_l):
    for k in range(BPW // LANES):
      mv = maskf_v[l, pl.ds(k * LANES, LANES)]
      for i in range(LANES):
        bcol = k * LANES + i
        m = mv[i]
        bvec = jnp.full((LANES,), bcol, jnp.int32)
        for e0 in range(COLV):
          v = rows[bcol, pl.ds(e0 * LANES, LANES)]
          plsc.store_scatter(out_l, [iota + (e0 * LANES), bvec], v * m)

  fire_gather(0, rows0, gsem0)
  fire_gather(1, rows1, gsem1)

  def pair_loop(lp, carry):
    l0 = 2 * lp
    wait_gather(rows0, gsem0)

    @pl.when(lp > 0)
    def _():
      wait_writes(out0, wsem0)
    transpose_mask(l0, rows0, out0)

    @pl.when(l0 + 2 < L)
    def _():
      fire_gather(l0 + 2, rows0, gsem0)
    fire_writes(l0, out0, wsem0)
    wait_gather(rows1, gsem1)

    @pl.when(lp > 0)
    def _():
      wait_writes(out1, wsem1)
    transpose_mask(l0 + 1, rows1, out1)

    @pl.when(l0 + 3 < L)
    def _():
      fire_gather(l0 + 3, rows1, gsem1)
    fire_writes(l0 + 1, out1, wsem1)
    return carry
  lax.fori_loop(0, L // 2, pair_loop, 0)

  wait_writes(out0, wsem0)
  wait_writes(out1, wsem1)

  def fut_gather(c, rows, gsem):
    pltpu.async_copy(table_h.at[fidx_v.at[c]], rows, gsem)

  def fut_wait_gather(rows, gsem):
    pltpu.make_async_copy(table_h.at[pl.ds(0, FCHUNK)], rows, gsem).wait()

  def fut_write(c, rows, wsem):
    pltpu.async_copy(rows, out_fut_h.at[pl.ds(basef + c * FCHUNK, FCHUNK)],
                     wsem)

  def fut_wait_write(rows, wsem):
    pltpu.make_async_copy(rows, out_fut_h.at[pl.ds(0, FCHUNK)], wsem).wait()

  fut_gather(0, rows0, gsem0)
  fut_gather(1, rows1, gsem1)
  fut_wait_gather(rows0, gsem0)
  fut_write(0, rows0, wsem0)
  fut_wait_gather(rows1, gsem1)
  fut_write(1, rows1, wsem1)
  fut_wait_write(rows0, wsem0)
  fut_gather(2, rows0, gsem0)
  fut_wait_write(rows1, wsem1)
  fut_gather(3, rows1, gsem1)
  fut_wait_gather(rows0, gsem0)
  fut_write(2, rows0, wsem0)
  fut_wait_gather(rows1, gsem1)
  fut_write(3, rows1, wsem1)
  fut_wait_write(rows0, wsem0)
  fut_wait_write(rows1, wsem1)


@jax.jit
def _run(sem_flat, tt_flat, msk_flat, semf_flat, ttf_flat, table):
  mesh = plsc.VectorSubcoreMesh(core_axis_name="c", subcore_axis_name="s",
                                num_cores=NC, num_subcores=NS)
  f = pl.kernel(
      _sc_body,
      out_type=[
          jax.ShapeDtypeStruct((L, ET, BT, 8, 128), jnp.float32),
          jax.ShapeDtypeStruct((NFUT, EMB_DIM), jnp.float32),
      ],
      mesh=mesh,
      scratch_types=[
          pltpu.VMEM((SLAB_PAD,), jnp.int32),
          pltpu.VMEM((SLAB_PAD,), jnp.int32),
          pltpu.VMEM((SLAB_PAD,), jnp.int32),
          pltpu.VMEM((L, BPW), jnp.int32),
          pltpu.VMEM((L, BPW), jnp.float32),
          pltpu.VMEM((FUT_CHUNKS, FCHUNK), jnp.int32),
          pltpu.VMEM((BPW, EMB_DIM), jnp.float32),
          pltpu.VMEM((BPW, EMB_DIM), jnp.float32),
          pltpu.VMEM((EMB_DIM, OUTW), jnp.float32),
          pltpu.VMEM((EMB_DIM, OUTW), jnp.float32),
          pltpu.SemaphoreType.DMA,
          pltpu.SemaphoreType.DMA,
          pltpu.SemaphoreType.DMA,
          pltpu.SemaphoreType.DMA,
      ],
      compiler_params=pltpu.CompilerParams(use_tc_tiling_on_sc=False,
                                           needs_layout_passes=False),
  )
  return f(sem_flat, tt_flat, msk_flat, semf_flat, ttf_flat, table)


def kernel(sem_ids, token_type_ids, seq_mask, sem_ids_fut, token_type_ids_fut,
           table):
  sem_flat = sem_ids.reshape(-1).astype(jnp.int32)
  tt_flat = token_type_ids.reshape(-1).astype(jnp.int32)
  msk_flat = seq_mask.reshape(-1).astype(jnp.int32)
  semf_flat = sem_ids_fut.reshape(-1).astype(jnp.int32)
  ttf_flat = token_type_ids_fut.reshape(-1).astype(jnp.int32)
  out5, out_fut = _run(sem_flat, tt_flat, msk_flat, semf_flat, ttf_flat,
                       table.astype(jnp.float32))
  out_seq = out5.transpose(2, 4, 0, 1, 3).reshape(B, L, EMB_DIM)
  return (out_seq, out_fut.reshape(B, LF, EMB_DIM))


# speedup vs baseline: 1.0868x; 1.0750x over previous
import functools

import jax
import jax.numpy as jnp
from jax import lax
from jax.experimental import pallas as pl
from jax.experimental.pallas import tpu as pltpu
from jax.experimental.pallas import tpu_sc as plsc

NUM_EMB = 100000
SEM_DIM = 4
EMB_DIM = 64
PAD = NUM_EMB * SEM_DIM

B, L, LF = 4096, 200, 4
NSEQ = B * L
NFUT = B * LF

NC, NS, LANES = 2, 16, 16
NW = NC * NS

SEQ_PER_W = NSEQ // NW
BPW = B // NW
FUT_PER_W = NFUT // NW

ET = EMB_DIM // 8
BT = B // 128
COLV = EMB_DIM // LANES
OUTW = 129

FCHUNK = 128
FUT_CHUNKS = FUT_PER_W // FCHUNK

SLAB = 1600
SLAB_PAD = SLAB + LANES
NSLABS = SEQ_PER_W // SLAB
SLAB_SEQS = SLAB // L
VECS_PER_SEQ = 13


def _idx_vec(sem_v, tt_v, off, use_mask, msk_v):
  s = sem_v[pl.ds(off, LANES)]
  t = tt_v[pl.ds(off, LANES)]
  tc = jnp.clip(t, 0, SEM_DIM - 1)
  sc = jnp.clip(s, 0, NUM_EMB - 1)
  idx = tc * NUM_EMB + sc
  keep = (s >= 0) & (s < NUM_EMB)
  if use_mask:
    m = msk_v[pl.ds(off, LANES)]
    keep = keep & (m != 0)
  mf = jnp.where(keep, jnp.float32(1.0), jnp.float32(0.0))
  return idx, mf


NSLOT = 4


def _sc_body(sem_h, tt_h, msk_h, semf_h, ttf_h, table_h,
             out5_h, out_fut_h,
             sem_v, tt_v, msk_v, idx_v, maskf_v, fidx_v,
             rows_all, out_all, gsems, wsems):
  wid = lax.axis_index("s") * NC + lax.axis_index("c")
  base = wid * SEQ_PER_W
  basef = wid * FUT_PER_W
  iota = lax.iota(jnp.int32, LANES)

  def slab_loop(sidx, carry):
    soff = base + sidx * SLAB
    pltpu.sync_copy(sem_h.at[pl.ds(soff, SLAB)], sem_v.at[pl.ds(0, SLAB)])
    pltpu.sync_copy(tt_h.at[pl.ds(soff, SLAB)], tt_v.at[pl.ds(0, SLAB)])
    pltpu.sync_copy(msk_h.at[pl.ds(soff, SLAB)], msk_v.at[pl.ds(0, SLAB)])

    def seq_loop(c, inner):
      bcol = sidx * SLAB_SEQS + c
      bvec = jnp.full((LANES,), bcol, jnp.int32)
      for u in range(VECS_PER_SEQ):
        idx, mf = _idx_vec(sem_v, tt_v, c * L + u * LANES, True, msk_v)
        lrow = iota + (u * LANES)
        if u < VECS_PER_SEQ - 1:
          plsc.store_scatter(idx_v, [lrow, bvec], idx)
          plsc.store_scatter(maskf_v, [lrow, bvec], mf)
        else:
          tail = iota < (L - (VECS_PER_SEQ - 1) * LANES)
          plsc.store_scatter(idx_v, [lrow, bvec], idx, mask=tail)
          plsc.store_scatter(maskf_v, [lrow, bvec], mf, mask=tail)
      return inner
    lax.fori_loop(0, SLAB_SEQS, seq_loop, 0)
    return carry
  lax.fori_loop(0, NSLABS, slab_loop, 0)

  pltpu.sync_copy(semf_h.at[pl.ds(basef, FUT_PER_W)],
                  sem_v.at[pl.ds(0, FUT_PER_W)])
  pltpu.sync_copy(ttf_h.at[pl.ds(basef, FUT_PER_W)],
                  tt_v.at[pl.ds(0, FUT_PER_W)])
  for c in range(FUT_CHUNKS):
    for u in range(FCHUNK // LANES):
      idx, _ = _idx_vec(sem_v, tt_v, c * FCHUNK + u * LANES, False, None)
      fidx_v[c, pl.ds(u * LANES, LANES)] = idx

  def fire_gather(l, rows, gsem):
    pltpu.async_copy(table_h.at[idx_v.at[l]], rows, gsem)

  def wait_gather(rows, gsem):
    pltpu.make_async_copy(table_h.at[pl.ds(0, BPW)], rows, gsem).wait()

  def fire_writes(l, out_l, wsem):
    for et in range(ET):
      pltpu.async_copy(out_l.at[pl.ds(et * 8, 8), pl.ds(0, 128)],
                       out5_h.at[l, et, wid], wsem)

  def wait_writes(out_l, wsem):
    for et in range(ET):
      pltpu.make_async_copy(out_l.at[pl.ds(et * 8, 8), pl.ds(0, 128)],
                            out5_h.at[0, et, 0], wsem).wait()

  def transpose_mask(l, rows, out_l):
    for k in range(BPW // LANES):
      mv = maskf_v[l, pl.ds(k * LANES, LANES)]
      for i in range(LANES):
        bcol = k * LANES + i
        m = mv[i]
        bvec = jnp.full((LANES,), bcol, jnp.int32)
        for e0 in range(COLV):
          v = rows[bcol, pl.ds(e0 * LANES, LANES)]
          plsc.store_scatter(out_l, [iota + (e0 * LANES), bvec], v * m)

  for s in range(NSLOT):
    fire_gather(s, rows_all.at[s], gsems.at[s])

  def l_loop(l, carry):
    s = lax.rem(l, NSLOT)
    rows = rows_all.at[s]
    out_l = out_all.at[s]
    gsem = gsems.at[s]
    wsem = wsems.at[s]
    wait_gather(rows, gsem)

    @pl.when(l >= NSLOT)
    def _():
      wait_writes(out_l, wsem)
    transpose_mask(l, rows, out_l)

    @pl.when(l + NSLOT < L)
    def _():
      fire_gather(l + NSLOT, rows, gsem)
    fire_writes(l, out_l, wsem)
    return carry
  lax.fori_loop(0, L, l_loop, 0)

  for s in range(NSLOT):
    wait_writes(out_all.at[s], wsems.at[s])

  def fut_gather(c, rows, gsem):
    pltpu.async_copy(table_h.at[fidx_v.at[c]], rows, gsem)

  def fut_wait_gather(rows, gsem):
    pltpu.make_async_copy(table_h.at[pl.ds(0, FCHUNK)], rows, gsem).wait()

  def fut_write(c, rows, wsem):
    pltpu.async_copy(rows, out_fut_h.at[pl.ds(basef + c * FCHUNK, FCHUNK)],
                     wsem)

  def fut_wait_write(rows, wsem):
    pltpu.make_async_copy(rows, out_fut_h.at[pl.ds(0, FCHUNK)], wsem).wait()

  for c in range(FUT_CHUNKS):
    fut_gather(c, rows_all.at[c], gsems.at[c])
  for c in range(FUT_CHUNKS):
    fut_wait_gather(rows_all.at[c], gsems.at[c])
    fut_write(c, rows_all.at[c], wsems.at[c])
  for c in range(FUT_CHUNKS):
    fut_wait_write(rows_all.at[c], wsems.at[c])


@jax.jit
def _run(sem_flat, tt_flat, msk_flat, semf_flat, ttf_flat, table):
  mesh = plsc.VectorSubcoreMesh(core_axis_name="c", subcore_axis_name="s",
                                num_cores=NC, num_subcores=NS)
  f = pl.kernel(
      _sc_body,
      out_type=[
          jax.ShapeDtypeStruct((L, ET, BT, 8, 128), jnp.float32),
          jax.ShapeDtypeStruct((NFUT, EMB_DIM), jnp.float32),
      ],
      mesh=mesh,
      scratch_types=[
          pltpu.VMEM((SLAB_PAD,), jnp.int32),
          pltpu.VMEM((SLAB_PAD,), jnp.int32),
          pltpu.VMEM((SLAB_PAD,), jnp.int32),
          pltpu.VMEM((L, BPW), jnp.int32),
          pltpu.VMEM((L, BPW), jnp.float32),
          pltpu.VMEM((FUT_CHUNKS, FCHUNK), jnp.int32),
          pltpu.VMEM((NSLOT, BPW, EMB_DIM), jnp.float32),
          pltpu.VMEM((NSLOT, EMB_DIM, OUTW), jnp.float32),
          pltpu.SemaphoreType.DMA((NSLOT,)),
          pltpu.SemaphoreType.DMA((NSLOT,)),
      ],
      compiler_params=pltpu.CompilerParams(use_tc_tiling_on_sc=False,
                                           needs_layout_passes=False),
  )
  return f(sem_flat, tt_flat, msk_flat, semf_flat, ttf_flat, table)


def kernel(sem_ids, token_type_ids, seq_mask, sem_ids_fut, token_type_ids_fut,
           table):
  sem_flat = sem_ids.reshape(-1).astype(jnp.int32)
  tt_flat = token_type_ids.reshape(-1).astype(jnp.int32)
  msk_flat = seq_mask.reshape(-1).astype(jnp.int32)
  semf_flat = sem_ids_fut.reshape(-1).astype(jnp.int32)
  ttf_flat = token_type_ids_fut.reshape(-1).astype(jnp.int32)
  out5, out_fut = _run(sem_flat, tt_flat, msk_flat, semf_flat, ttf_flat,
                       table.astype(jnp.float32))
  out_seq = out5.transpose(2, 4, 0, 1, 3).reshape(B, L, EMB_DIM)
  return (out_seq, out_fut.reshape(B, LF, EMB_DIM))
